# baseline (speedup 1.0000x reference)
import jax
import jax.numpy as jnp
from jax import lax
from jax.experimental import pallas as pl
from jax.experimental.pallas import tpu as pltpu
from jax.experimental.pallas import tpu_sc as plsc

_B = 512
_G = 20000
_TF = 1600
_WM = 4
_H = _TF * _WM
_FANIN = 16
_L = 16
_NC = 2
_NS = 16
_NW = _NC * _NS
_BPW = _B // _NW
_ROWS = 4
_NPASS = _BPW // _ROWS
_TG = _TF // _L
_CH = 2
_NCHUNK = _TG // _CH
_CHUNK_W = _CH * _WM * _FANIN * _L
_TOTAL = _NPASS * _NCHUNK


def _leaky(x):
    return jnp.where(x >= 0, x, x * 0.01)


def _body(feat, idxh, wh, b1h, b2h, w2h, zout,
          rows_v, zbuf, b1v, b2v, w2v, idxv, wv, spi, spw,
          semi0, semi1, semw0, semw1):
    c = lax.axis_index("c")
    s = lax.axis_index("s")
    wid = s * _NC + c
    bbase = wid * _BPW
    semi = (semi0, semi1)
    semw = (semw0, semw1)

    @pl.when(s == 0)
    def _():
        pltpu.sync_copy(idxh, spi)
        pltpu.sync_copy(wh, spw)

    pltpu.sync_copy(b1h, b1v)
    pltpu.sync_copy(b2h, b2v)
    pltpu.sync_copy(w2h, w2v)
    plsc.subcore_barrier()

    def start(gi, slot):
        ch = lax.rem(gi, _NCHUNK)
        off = ch * _CHUNK_W
        pltpu.async_copy(spi.at[pl.ds(off, _CHUNK_W)], idxv.at[slot],
                         semi[slot])
        pltpu.async_copy(spw.at[pl.ds(off, _CHUNK_W)], wv.at[slot],
                         semw[slot])

    def wait(slot):
        pltpu.make_async_copy(spi.at[pl.ds(0, _CHUNK_W)], idxv.at[slot],
                              semi[slot]).wait()
        pltpu.make_async_copy(spw.at[pl.ds(0, _CHUNK_W)], wv.at[slot],
                              semw[slot]).wait()

    start(jnp.int32(0), 0)

    def outer(g2, _):
        for slot in (0, 1):
            gi = g2 * 2 + slot
            ch = lax.rem(gi, _NCHUNK)
            p = lax.div(gi, _NCHUNK)
            b0 = bbase + p * _ROWS

            @pl.when(jnp.logical_and(ch == 0, gi > 0))
            def _():
                pltpu.sync_copy(
                    zbuf, zout.at[pl.ds(bbase + (p - 1) * _ROWS, _ROWS)])

            @pl.when(ch == 0)
            def _():
                pltpu.sync_copy(feat.at[pl.ds(b0, _ROWS)], rows_v)

            @pl.when(gi + 1 < _TOTAL)
            def _():
                start(gi + 1, 1 - slot)

            wait(slot)

            @plsc.parallel_loop(0, _CH, 1, unroll=2)
            def tg_body(tl, *, slot=slot, ch=ch):
                tg = ch * _CH + tl
                zaccs = [None] * _ROWS
                for m in range(_WM):
                    eoff = (tl * _WM + m) * (_FANIN * _L)
                    noff = (tg * _WM + m) * _L
                    b1vec = b1v[pl.ds(noff, _L)]
                    accs = [b1vec] * _ROWS
                    for k in range(_FANIN):
                        iv = idxv[slot, pl.ds(eoff + k * _L, _L)]
                        wvk = wv[slot, pl.ds(eoff + k * _L, _L)]
                        for b in range(_ROWS):
                            bvec = jnp.full((_L,), b, dtype=jnp.int32)
                            g = plsc.load_gather(rows_v, [bvec, iv])
                            accs[b] = accs[b] + g * wvk
                    w2vec = w2v[pl.ds(noff, _L)]
                    for b in range(_ROWS):
                        contrib = _leaky(accs[b]) * w2vec
                        if m == 0:
                            zaccs[b] = b2v[pl.ds(tg * _L, _L)] + contrib
                        else:
                            zaccs[b] = zaccs[b] + contrib
                for b in range(_ROWS):
                    zbuf[b, pl.ds(tg * _L, _L)] = _leaky(zaccs[b])
        return 0

    lax.fori_loop(0, _TOTAL // 2, outer, 0)
    pltpu.sync_copy(zbuf, zout.at[pl.ds(bbase + (_NPASS - 1) * _ROWS, _ROWS)])


_sc_call = pl.kernel(
    _body,
    out_type=jax.ShapeDtypeStruct((_B, _TF), jnp.float32),
    mesh=plsc.VectorSubcoreMesh(core_axis_name="c", subcore_axis_name="s"),
    compiler_params=pltpu.CompilerParams(needs_layout_passes=False),
    scratch_types=[
        pltpu.VMEM((_ROWS, _G), jnp.float32),
        pltpu.VMEM((_ROWS, _TF), jnp.float32),
        pltpu.VMEM((_H,), jnp.float32),
        pltpu.VMEM((_TF,), jnp.float32),
        pltpu.VMEM((_H,), jnp.float32),
        pltpu.VMEM((2, _CHUNK_W), jnp.int32),
        pltpu.VMEM((2, _CHUNK_W), jnp.float32),
        pltpu.VMEM_SHARED((_H * _FANIN,), jnp.int32),
        pltpu.VMEM_SHARED((_H * _FANIN,), jnp.float32),
        pltpu.SemaphoreType.DMA,
        pltpu.SemaphoreType.DMA,
        pltpu.SemaphoreType.DMA,
        pltpu.SemaphoreType.DMA,
    ],
)


def kernel(features, src1, dst1, w1, b1, src2, dst2, w2, b2):
    idx_arr = (src1.astype(jnp.int32)
               .reshape(_TG, _L, _WM, _FANIN)
               .transpose(0, 2, 3, 1).reshape(-1))
    w_arr = w1.reshape(_TG, _L, _WM, _FANIN).transpose(0, 2, 3, 1).reshape(-1)
    b1r = b1.reshape(_TG, _L, _WM).transpose(0, 2, 1).reshape(-1)
    w2r = w2.reshape(_TG, _L, _WM).transpose(0, 2, 1).reshape(-1)
    return _sc_call(features, idx_arr, w_arr, b1r, b2, w2r)


# speedup vs baseline: 2.0156x; 2.0156x over previous
import jax
import jax.numpy as jnp
from jax import lax
from jax.experimental import pallas as pl
from jax.experimental.pallas import tpu as pltpu
from jax.experimental.pallas import tpu_sc as plsc

_B = 512
_G = 20000
_TF = 1600
_WM = 4
_H = _TF * _WM
_FANIN = 16
_L = 16
_NC = 2
_NS = 16
_NW = _NC * _NS
_BPW = _B // _NW
_ROWS = 4
_NPASS = _BPW // _ROWS
_TG = _TF // _L
_CH = 4
_NCHUNK = _TG // _CH
_KP = _FANIN // 2
_CHUNK_WI = _CH * _WM * _KP * _L
_CHUNK_WW = _CH * _WM * _FANIN * _L
_TOTAL = _NPASS * _NCHUNK


def _leaky(x):
    return jnp.where(x >= 0, x, x * 0.01)


def _body(feat, idxh, wh, b1h, b2h, w2h, zout,
          rows_v, zbuf, b1v, b2v, w2v, idxv, wv, spi, spw,
          semi0, semi1, semw0, semw1):
    c = lax.axis_index("c")
    s = lax.axis_index("s")
    wid = s * _NC + c
    bbase = wid * _BPW
    semi = (semi0, semi1)
    semw = (semw0, semw1)

    @pl.when(s == 0)
    def _():
        pltpu.sync_copy(idxh, spi)
        pltpu.sync_copy(wh, spw)

    pltpu.sync_copy(b1h, b1v)
    pltpu.sync_copy(b2h, b2v)
    pltpu.sync_copy(w2h, w2v)
    plsc.subcore_barrier()

    def start(gi, slot):
        ch = lax.rem(gi, _NCHUNK)
        pltpu.async_copy(spi.at[pl.ds(ch * _CHUNK_WI, _CHUNK_WI)],
                         idxv.at[slot], semi[slot])
        pltpu.async_copy(spw.at[pl.ds(ch * _CHUNK_WW, _CHUNK_WW)],
                         wv.at[slot], semw[slot])

    def wait(slot):
        pltpu.make_async_copy(spi.at[pl.ds(0, _CHUNK_WI)], idxv.at[slot],
                              semi[slot]).wait()
        pltpu.make_async_copy(spw.at[pl.ds(0, _CHUNK_WW)], wv.at[slot],
                              semw[slot]).wait()

    start(jnp.int32(0), 0)

    def outer(g2, _):
        for slot in (0, 1):
            gi = g2 * 2 + slot
            ch = lax.rem(gi, _NCHUNK)
            p = lax.div(gi, _NCHUNK)
            b0 = bbase + p * _ROWS

            @pl.when(jnp.logical_and(ch == 0, gi > 0))
            def _():
                pltpu.sync_copy(
                    zbuf, zout.at[pl.ds(bbase + (p - 1) * _ROWS, _ROWS)])

            @pl.when(ch == 0)
            def _():
                pltpu.sync_copy(feat.at[pl.ds(b0, _ROWS)], rows_v)

            @pl.when(gi + 1 < _TOTAL)
            def _():
                start(gi + 1, 1 - slot)

            wait(slot)

            def tg_body(tl, _, *, slot=slot, ch=ch):
                tg = ch * _CH + tl
                zaccs = [None] * _ROWS
                for m in range(_WM):
                    eoff = (tl * _WM + m) * (_FANIN * _L)
                    eoffi = (tl * _WM + m) * (_KP * _L)
                    noff = (tg * _WM + m) * _L
                    b1vec = b1v[pl.ds(noff, _L)]
                    accs = [b1vec] * _ROWS
                    for j in range(_KP):
                        pv = idxv[slot, pl.ds(eoffi + j * _L, _L)]
                        iv0 = lax.bitwise_and(pv, jnp.int32(0xFFFF))
                        iv1 = lax.shift_right_logical(pv, jnp.int32(16))
                        for k, iv in ((2 * j, iv0), (2 * j + 1, iv1)):
                            wvk = wv[slot, pl.ds(eoff + k * _L, _L)]
                            for b in range(_ROWS):
                                bvec = jnp.full((_L,), b, dtype=jnp.int32)
                                g = plsc.load_gather(rows_v, [bvec, iv])
                                accs[b] = accs[b] + g * wvk
                    w2vec = w2v[pl.ds(noff, _L)]
                    for b in range(_ROWS):
                        contrib = _leaky(accs[b]) * w2vec
                        if m == 0:
                            zaccs[b] = b2v[pl.ds(tg * _L, _L)] + contrib
                        else:
                            zaccs[b] = zaccs[b] + contrib
                for b in range(_ROWS):
                    zbuf[b, pl.ds(tg * _L, _L)] = _leaky(zaccs[b])
                return 0

            lax.fori_loop(0, _CH, tg_body, 0)
        return 0

    lax.fori_loop(0, _TOTAL // 2, outer, 0)
    pltpu.sync_copy(zbuf, zout.at[pl.ds(bbase + (_NPASS - 1) * _ROWS, _ROWS)])


_sc_call = pl.kernel(
    _body,
    out_type=jax.ShapeDtypeStruct((_B, _TF), jnp.float32),
    mesh=plsc.VectorSubcoreMesh(core_axis_name="c", subcore_axis_name="s"),
    compiler_params=pltpu.CompilerParams(needs_layout_passes=False),
    scratch_types=[
        pltpu.VMEM((_ROWS, _G), jnp.float32),
        pltpu.VMEM((_ROWS, _TF), jnp.float32),
        pltpu.VMEM((_H,), jnp.float32),
        pltpu.VMEM((_TF,), jnp.float32),
        pltpu.VMEM((_H,), jnp.float32),
        pltpu.VMEM((2, _CHUNK_WI), jnp.int32),
        pltpu.VMEM((2, _CHUNK_WW), jnp.float32),
        pltpu.VMEM_SHARED((_H * _KP,), jnp.int32),
        pltpu.VMEM_SHARED((_H * _FANIN,), jnp.float32),
        pltpu.SemaphoreType.DMA,
        pltpu.SemaphoreType.DMA,
        pltpu.SemaphoreType.DMA,
        pltpu.SemaphoreType.DMA,
    ],
)


def kernel(features, src1, dst1, w1, b1, src2, dst2, w2, b2):
    eye = jnp.eye(_L, dtype=jnp.float32)
    hp = jax.lax.Precision.HIGHEST
    idx_t = jnp.einsum('tje,jl->tel', src1.astype(jnp.float32)
                       .reshape(_TG, _L, _WM * _FANIN), eye,
                       precision=hp).astype(jnp.int32)
    idx_t = idx_t.reshape(_TG, _WM, _FANIN, _L)
    idx_arr = (idx_t[:, :, 0::2, :]
               + (idx_t[:, :, 1::2, :] << 16)).reshape(-1)
    w_arr = jnp.einsum('tje,jl->tel', w1.reshape(_TG, _L, _WM * _FANIN),
                       eye, precision=hp).reshape(-1)
    b1r = jnp.einsum('tjm,jl->tml', b1.reshape(_TG, _L, _WM), eye,
                     precision=hp).reshape(-1)
    w2r = jnp.einsum('tjm,jl->tml', w2.reshape(_TG, _L, _WM), eye,
                     precision=hp).reshape(-1)
    return _sc_call(features, idx_arr, w_arr, b1r, b2, w2r)
